# Initial kernel scaffold; baseline (speedup 1.0000x reference)
import jax
import jax.numpy as jnp
from jax.experimental import pallas as pl


def kernel(node, batch_index, emb0, emb1, emb2, W1, b1, W2, b2):
    raise NotImplementedError("write your pallas kernel here")


# speedup vs baseline: 2.6584x; 2.6584x over previous
import functools

import jax
import jax.numpy as jnp
from jax import lax
from jax.experimental import pallas as pl
from jax.experimental.pallas import tpu as pltpu
from jax.experimental.pallas import tpu_sc as plsc

_N = 320000
_H = 128
_V = 100
_NUM_SEG = 10000
_SEG_V = _NUM_SEG * _V
_NC = 2
_NS = 16
_NW = _NC * _NS
_CHUNK = _N // _NW
_TS = 62720
_TSQ = _TS // 4
_ROWP = _NS * _TS
_G = 79


@functools.lru_cache(maxsize=1)
def _sc_histogram_build():
    mesh = plsc.VectorSubcoreMesh(core_axis_name="c", subcore_axis_name="s")

    @functools.partial(
        pl.kernel,
        out_type=jax.ShapeDtypeStruct((6 * _ROWP,), jnp.float32),
        mesh=mesh,
        scratch_types=[
            pltpu.VMEM((_CHUNK,), jnp.int32),
            pltpu.VMEM((_CHUNK,), jnp.int32),
            pltpu.VMEM((_G, 128), jnp.int32),
            pltpu.VMEM((128,), jnp.float32),
            pltpu.VMEM((_TSQ,), jnp.float32),
            pltpu.VMEM((_TSQ,), jnp.float32),
            pltpu.VMEM_SHARED((_ROWP,), jnp.float32),
        ],
    )
    def sc_hist(bat_hbm, c0_hbm, c1_hbm, c2_hbm, out_hbm,
                bat_v, col_v, idx_v, ones_v, zeros_v, bounce_v, hist_sp):
        cid = lax.axis_index("c")
        sid = lax.axis_index("s")
        wid = cid * _NS + sid
        base = wid * _CHUNK
        toff = sid * _TS

        def _zfill(i, _):
            zeros_v[pl.ds(i * 16, 16)] = jnp.zeros((16,), jnp.float32)
            return _
        lax.fori_loop(0, _TSQ // 16, _zfill, None)
        for j in range(8):
            ones_v[pl.ds(j * 16, 16)] = jnp.ones((16,), jnp.float32)
        for j in range(1, 8):
            idx_v[_G - 1, pl.ds(j * 16, 16)] = jnp.full((16,), _SEG_V, jnp.int32)

        pltpu.sync_copy(bat_hbm.at[pl.ds(base, _CHUNK)], bat_v)
        for q in range(4):
            pltpu.sync_copy(zeros_v, hist_sp.at[pl.ds(toff + q * _TSQ, _TSQ)])
        plsc.subcore_barrier()

        for k, col_hbm in enumerate((c0_hbm, c1_hbm, c2_hbm)):
            pltpu.sync_copy(col_hbm.at[pl.ds(base, _CHUNK)], col_v)

            def _fill(g, _):
                nb = g * 128
                for j in range(8):
                    b16 = bat_v[pl.ds(nb + j * 16, 16)]
                    a16 = col_v[pl.ds(nb + j * 16, 16)]
                    idx_v[g, pl.ds(j * 16, 16)] = b16 * _V + a16
                return _
            lax.fori_loop(0, _G - 1, _fill, None)
            b16 = bat_v[pl.ds((_G - 1) * 128, 16)]
            a16 = col_v[pl.ds((_G - 1) * 128, 16)]
            idx_v[_G - 1, pl.ds(0, 16)] = b16 * _V + a16

            def _scat(g, _):
                pltpu.sync_copy(ones_v, hist_sp.at[idx_v.at[g]], add=True)
                return _
            lax.fori_loop(0, _G, _scat, None)

            plsc.subcore_barrier()
            row = cid * 3 + k
            for q in range(4):
                off = toff + q * _TSQ
                pltpu.sync_copy(hist_sp.at[pl.ds(off, _TSQ)], bounce_v)
                pltpu.sync_copy(bounce_v,
                                out_hbm.at[pl.ds(row * _ROWP + off, _TSQ)])
                if k < 2:
                    pltpu.sync_copy(zeros_v, hist_sp.at[pl.ds(off, _TSQ)])
            plsc.subcore_barrier()

    return sc_hist


_BLK = 1024


def _tc_mlp_body(c_ref, e_ref, w1_ref, b1_ref, w2_ref, b2_ref, out_ref):
    C = c_ref[...]
    E = e_ref[...]
    acc = lax.dot(C[0], E[0], preferred_element_type=jnp.float32)
    for i in range(1, 6):
        acc += lax.dot(C[i], E[i], preferred_element_type=jnp.float32)
    n = jnp.sum(C[0] + C[3], axis=1)
    mean = acc / jnp.maximum(n, 1.0)[:, None]
    h = lax.dot_general(mean, w1_ref[...],
                        (((1,), (1,)), ((), ())),
                        preferred_element_type=jnp.float32)
    h = jnp.maximum(h + b1_ref[...][None, :], 0.0)
    o = lax.dot_general(h, w2_ref[...],
                        (((1,), (1,)), ((), ())),
                        preferred_element_type=jnp.float32)
    out_ref[...] = o[:, 0] + b2_ref[0]


def _tc_mlp(C6, E6, W1, b1, W2, b2):
    grid = (_NUM_SEG + _BLK - 1) // _BLK
    return pl.pallas_call(
        _tc_mlp_body,
        grid=(grid,),
        in_specs=[
            pl.BlockSpec((6, _BLK, _V), lambda i: (0, i, 0)),
            pl.BlockSpec((6, _V, _H), lambda i: (0, 0, 0)),
            pl.BlockSpec((_H, _H), lambda i: (0, 0)),
            pl.BlockSpec((_H,), lambda i: (0,)),
            pl.BlockSpec((1, _H), lambda i: (0, 0)),
            pl.BlockSpec(memory_space=pltpu.SMEM),
        ],
        out_specs=pl.BlockSpec((_BLK,), lambda i: (i,)),
        out_shape=jax.ShapeDtypeStruct((_NUM_SEG,), jnp.float32),
    )(C6, E6, W1, b1, W2, b2)


def kernel(node, batch_index, emb0, emb1, emb2, W1, b1, W2, b2):
    c0 = node[:, 0]
    c1 = node[:, 1]
    c2 = node[:, 2]
    outC = _sc_histogram_build()(batch_index, c0, c1, c2)
    C6 = outC.reshape(6, _ROWP)[:, :_SEG_V].reshape(6, _NUM_SEG, _V)
    E6 = jnp.stack([emb0, emb1, emb2, emb0, emb1, emb2]) * (1.0 / 3.0)
    return _tc_mlp(C6, E6, W1, b1, W2, b2)


# speedup vs baseline: 26.2688x; 9.8815x over previous
import functools

import jax
import jax.numpy as jnp
from jax import lax
from jax.experimental import pallas as pl
from jax.experimental.pallas import tpu as pltpu
from jax.experimental.pallas import tpu_sc as plsc

_N = 320000
_H = 128
_V = 100
_VP = 128
_NUM_SEG = 10000
_SEG_VP = _NUM_SEG * _VP
_NC = 2
_NS = 16
_NW = _NC * _NS
_CHUNK = _N // _NW
_TS = _SEG_VP // _NS
_TSQ = _TS // 8
_NQ = _TS // _TSQ
_G = 79


@functools.lru_cache(maxsize=1)
def _sc_histogram_build():
    mesh = plsc.VectorSubcoreMesh(core_axis_name="c", subcore_axis_name="s")

    @functools.partial(
        pl.kernel,
        out_type=jax.ShapeDtypeStruct((6 * _SEG_VP,), jnp.float32),
        mesh=mesh,
        scratch_types=[
            pltpu.VMEM((_CHUNK,), jnp.int32),
            pltpu.VMEM((_CHUNK,), jnp.int32),
            pltpu.VMEM((_G, 128), jnp.int32),
            pltpu.VMEM((128,), jnp.float32),
            pltpu.VMEM((_TSQ,), jnp.float32),
            pltpu.VMEM((_TSQ,), jnp.float32),
            pltpu.VMEM_SHARED((_SEG_VP,), jnp.float32),
        ],
    )
    def sc_hist(bat_hbm, c0_hbm, c1_hbm, c2_hbm, out_hbm,
                bat_v, col_v, idx_v, ones_v, zeros_v, bounce_v, hist_sp):
        cid = lax.axis_index("c")
        sid = lax.axis_index("s")
        wid = cid * _NS + sid
        base = wid * _CHUNK
        toff = sid * _TS

        def _zfill(i, _):
            zeros_v[pl.ds(i * 16, 16)] = jnp.zeros((16,), jnp.float32)
            return _
        lax.fori_loop(0, _TSQ // 16, _zfill, None)
        for j in range(8):
            ones_v[pl.ds(j * 16, 16)] = jnp.ones((16,), jnp.float32)
        for j in range(1, 8):
            idx_v[_G - 1, pl.ds(j * 16, 16)] = jnp.full((16,), _VP - 1, jnp.int32)

        pltpu.sync_copy(bat_hbm.at[pl.ds(base, _CHUNK)], bat_v)
        for q in range(_NQ):
            pltpu.sync_copy(zeros_v, hist_sp.at[pl.ds(toff + q * _TSQ, _TSQ)])
        plsc.subcore_barrier()

        for k, col_hbm in enumerate((c0_hbm, c1_hbm, c2_hbm)):
            pltpu.sync_copy(col_hbm.at[pl.ds(base, _CHUNK)], col_v)

            def _fill(g, _):
                nb = g * 128
                for j in range(8):
                    b16 = bat_v[pl.ds(nb + j * 16, 16)]
                    a16 = col_v[pl.ds(nb + j * 16, 16)]
                    idx_v[g, pl.ds(j * 16, 16)] = b16 * _VP + a16
                return _
            lax.fori_loop(0, _G - 1, _fill, None)
            b16 = bat_v[pl.ds((_G - 1) * 128, 16)]
            a16 = col_v[pl.ds((_G - 1) * 128, 16)]
            idx_v[_G - 1, pl.ds(0, 16)] = b16 * _VP + a16

            def _scat(g, _):
                pltpu.sync_copy(ones_v, hist_sp.at[idx_v.at[g]], add=True)
                return _
            lax.fori_loop(0, _G, _scat, None)

            plsc.subcore_barrier()
            row = cid * 3 + k
            for q in range(_NQ):
                off = toff + q * _TSQ
                pltpu.sync_copy(hist_sp.at[pl.ds(off, _TSQ)], bounce_v)
                pltpu.sync_copy(bounce_v,
                                out_hbm.at[pl.ds(row * _SEG_VP + off, _TSQ)])
                if k < 2:
                    pltpu.sync_copy(zeros_v, hist_sp.at[pl.ds(off, _TSQ)])
            plsc.subcore_barrier()

    return sc_hist


_BLK = 1024


def _tc_mlp_body(c_ref, e_ref, w1_ref, b1_ref, w2_ref, b2_ref, out_ref):
    C = c_ref[...]
    E = e_ref[...]
    acc = lax.dot(C[0], E[0], preferred_element_type=jnp.float32)
    for i in range(1, 6):
        acc += lax.dot(C[i], E[i], preferred_element_type=jnp.float32)
    vmask = lax.broadcasted_iota(jnp.int32, (_BLK, _VP), 1) < _V
    cnt = jnp.where(vmask, C[0] + C[3], 0.0)
    n = jnp.sum(cnt, axis=1)
    mean = acc / jnp.maximum(n, 1.0)[:, None]
    h = lax.dot_general(mean, w1_ref[...],
                        (((1,), (1,)), ((), ())),
                        preferred_element_type=jnp.float32)
    h = jnp.maximum(h + b1_ref[...][None, :], 0.0)
    o = lax.dot_general(h, w2_ref[...],
                        (((1,), (1,)), ((), ())),
                        preferred_element_type=jnp.float32)
    out_ref[...] = o[:, 0] + b2_ref[0]


def _tc_mlp(C6, E6, W1, b1, W2, b2):
    grid = (_NUM_SEG + _BLK - 1) // _BLK
    return pl.pallas_call(
        _tc_mlp_body,
        grid=(grid,),
        in_specs=[
            pl.BlockSpec((6, _BLK, _VP), lambda i: (0, i, 0)),
            pl.BlockSpec((6, _VP, _H), lambda i: (0, 0, 0)),
            pl.BlockSpec((_H, _H), lambda i: (0, 0)),
            pl.BlockSpec((_H,), lambda i: (0,)),
            pl.BlockSpec((1, _H), lambda i: (0, 0)),
            pl.BlockSpec(memory_space=pltpu.SMEM),
        ],
        out_specs=pl.BlockSpec((_BLK,), lambda i: (i,)),
        out_shape=jax.ShapeDtypeStruct((_NUM_SEG,), jnp.float32),
    )(C6, E6, W1, b1, W2, b2)


def kernel(node, batch_index, emb0, emb1, emb2, W1, b1, W2, b2):
    c0 = node[:, 0]
    c1 = node[:, 1]
    c2 = node[:, 2]
    outC = _sc_histogram_build()(batch_index, c0, c1, c2)
    C6 = outC.reshape(6, _NUM_SEG, _VP)
    E3 = jnp.concatenate(
        [jnp.stack([emb0, emb1, emb2]) * (1.0 / 3.0),
         jnp.zeros((3, _VP - _V, _H), jnp.float32)], axis=1)
    E6 = jnp.concatenate([E3, E3], axis=0)
    return _tc_mlp(C6, E6, W1, b1, W2, b2)


# speedup vs baseline: 29.7212x; 1.1314x over previous
import functools

import jax
import jax.numpy as jnp
from jax import lax
from jax.experimental import pallas as pl
from jax.experimental.pallas import tpu as pltpu
from jax.experimental.pallas import tpu_sc as plsc

_N = 320000
_H = 128
_V = 100
_VP = 128
_NUM_SEG = 10000
_SEG_VP = _NUM_SEG * _VP
_NC = 2
_NS = 16
_NW = _NC * _NS
_CHUNK = _N // _NW
_TS = _SEG_VP // _NS
_TSQ = _TS // 8
_NQ = _TS // _TSQ
_G = 79


@functools.lru_cache(maxsize=1)
def _sc_histogram_build():
    mesh = plsc.VectorSubcoreMesh(core_axis_name="c", subcore_axis_name="s")

    @functools.partial(
        pl.kernel,
        out_type=jax.ShapeDtypeStruct((6 * _SEG_VP,), jnp.float32),
        mesh=mesh,
        scratch_types=[
            pltpu.VMEM((_CHUNK,), jnp.int32),
            pltpu.VMEM((_CHUNK,), jnp.int32),
            pltpu.VMEM((_G, 128), jnp.int32),
            pltpu.VMEM((128,), jnp.float32),
            pltpu.VMEM((_TSQ,), jnp.float32),
            pltpu.VMEM((_TSQ,), jnp.float32),
            pltpu.VMEM_SHARED((_SEG_VP,), jnp.float32),
            pltpu.SemaphoreType.DMA,
            pltpu.SemaphoreType.DMA,
        ],
    )
    def sc_hist(bat_hbm, c0_hbm, c1_hbm, c2_hbm, out_hbm,
                bat_v, col_v, idx_v, ones_v, zeros_v, bounce_v, hist_sp,
                sem_s, sem_o):
        cid = lax.axis_index("c")
        sid = lax.axis_index("s")
        wid = cid * _NS + sid
        base = wid * _CHUNK
        toff = sid * _TS

        def _zfill(i, _):
            zeros_v[pl.ds(i * 16, 16)] = jnp.zeros((16,), jnp.float32)
            return _
        lax.fori_loop(0, _TSQ // 16, _zfill, None)
        for j in range(8):
            ones_v[pl.ds(j * 16, 16)] = jnp.ones((16,), jnp.float32)
        for j in range(1, 8):
            idx_v[_G - 1, pl.ds(j * 16, 16)] = jnp.full((16,), _VP - 1, jnp.int32)

        pltpu.sync_copy(bat_hbm.at[pl.ds(base, _CHUNK)], bat_v)
        zd = [pltpu.async_copy(zeros_v, hist_sp.at[pl.ds(toff + q * _TSQ, _TSQ)],
                               sem_o) for q in range(_NQ)]
        for d in zd:
            d.wait()
        plsc.subcore_barrier()

        for k, col_hbm in enumerate((c0_hbm, c1_hbm, c2_hbm)):
            pltpu.sync_copy(col_hbm.at[pl.ds(base, _CHUNK)], col_v)

            def _fill(g, _):
                nb = g * 128
                for j in range(8):
                    b16 = bat_v[pl.ds(nb + j * 16, 16)]
                    a16 = col_v[pl.ds(nb + j * 16, 16)]
                    idx_v[g, pl.ds(j * 16, 16)] = b16 * _VP + a16
                return _
            lax.fori_loop(0, _G - 1, _fill, None)
            b16 = bat_v[pl.ds((_G - 1) * 128, 16)]
            a16 = col_v[pl.ds((_G - 1) * 128, 16)]
            idx_v[_G - 1, pl.ds(0, 16)] = b16 * _VP + a16

            sd = [pltpu.async_copy(ones_v, hist_sp.at[idx_v.at[g]],
                                   sem_s, add=True) for g in range(_G)]
            for d in sd:
                d.wait()

            plsc.subcore_barrier()
            row = cid * 3 + k
            for q in range(_NQ):
                off = toff + q * _TSQ
                pltpu.sync_copy(hist_sp.at[pl.ds(off, _TSQ)], bounce_v)
                d1 = pltpu.async_copy(
                    bounce_v, out_hbm.at[pl.ds(row * _SEG_VP + off, _TSQ)],
                    sem_o)
                if k < 2:
                    pltpu.sync_copy(zeros_v, hist_sp.at[pl.ds(off, _TSQ)])
                d1.wait()
            plsc.subcore_barrier()

    return sc_hist


_BLK = 1024


def _tc_mlp_body(c_ref, e_ref, w1_ref, b1_ref, w2_ref, b2_ref, out_ref):
    C = c_ref[...]
    E = e_ref[...]
    acc = lax.dot(C[0], E[0], preferred_element_type=jnp.float32)
    for i in range(1, 6):
        acc += lax.dot(C[i], E[i], preferred_element_type=jnp.float32)
    vmask = lax.broadcasted_iota(jnp.int32, (_BLK, _VP), 1) < _V
    cnt = jnp.where(vmask, C[0] + C[3], 0.0)
    n = jnp.sum(cnt, axis=1)
    mean = acc / jnp.maximum(n, 1.0)[:, None]
    h = lax.dot_general(mean, w1_ref[...],
                        (((1,), (1,)), ((), ())),
                        preferred_element_type=jnp.float32)
    h = jnp.maximum(h + b1_ref[...][None, :], 0.0)
    o = lax.dot_general(h, w2_ref[...],
                        (((1,), (1,)), ((), ())),
                        preferred_element_type=jnp.float32)
    out_ref[...] = o[:, 0] + b2_ref[0]


def _tc_mlp(C6, E6, W1, b1, W2, b2):
    grid = (_NUM_SEG + _BLK - 1) // _BLK
    return pl.pallas_call(
        _tc_mlp_body,
        grid=(grid,),
        in_specs=[
            pl.BlockSpec((6, _BLK, _VP), lambda i: (0, i, 0)),
            pl.BlockSpec((6, _VP, _H), lambda i: (0, 0, 0)),
            pl.BlockSpec((_H, _H), lambda i: (0, 0)),
            pl.BlockSpec((_H,), lambda i: (0,)),
            pl.BlockSpec((1, _H), lambda i: (0, 0)),
            pl.BlockSpec(memory_space=pltpu.SMEM),
        ],
        out_specs=pl.BlockSpec((_BLK,), lambda i: (i,)),
        out_shape=jax.ShapeDtypeStruct((_NUM_SEG,), jnp.float32),
    )(C6, E6, W1, b1, W2, b2)


def kernel(node, batch_index, emb0, emb1, emb2, W1, b1, W2, b2):
    c0 = node[:, 0]
    c1 = node[:, 1]
    c2 = node[:, 2]
    outC = _sc_histogram_build()(batch_index, c0, c1, c2)
    C6 = outC.reshape(6, _NUM_SEG, _VP)
    E3 = jnp.concatenate(
        [jnp.stack([emb0, emb1, emb2]) * (1.0 / 3.0),
         jnp.zeros((3, _VP - _V, _H), jnp.float32)], axis=1)
    E6 = jnp.concatenate([E3, E3], axis=0)
    return _tc_mlp(C6, E6, W1, b1, W2, b2)


# speedup vs baseline: 33.0185x; 1.1109x over previous
import functools

import jax
import jax.numpy as jnp
from jax import lax
from jax.experimental import pallas as pl
from jax.experimental.pallas import tpu as pltpu
from jax.experimental.pallas import tpu_sc as plsc

_N = 320000
_H = 128
_V = 100
_VP = 128
_NUM_SEG = 10000
_SEG_VP = _NUM_SEG * _VP
_NC = 2
_NS = 16
_NW = _NC * _NS
_CHUNK = _N // _NW
_TS = _SEG_VP // _NS
_TSQ = 4000
_NQ = _TS // _TSQ
_G = 79


@functools.lru_cache(maxsize=1)
def _sc_histogram_build():
    mesh = plsc.VectorSubcoreMesh(core_axis_name="c", subcore_axis_name="s")

    @functools.partial(
        pl.kernel,
        out_type=jax.ShapeDtypeStruct((6 * _SEG_VP,), jnp.float32),
        mesh=mesh,
        scratch_types=[
            pltpu.VMEM((_CHUNK,), jnp.int32),
            pltpu.VMEM((_CHUNK,), jnp.int32),
            pltpu.VMEM((_G, 128), jnp.int32),
            pltpu.VMEM((128,), jnp.float32),
            pltpu.VMEM((_TSQ,), jnp.float32),
            pltpu.VMEM((_TSQ,), jnp.float32),
            pltpu.VMEM((_TSQ,), jnp.float32),
            pltpu.VMEM_SHARED((_SEG_VP,), jnp.float32),
            pltpu.SemaphoreType.DMA,
            pltpu.SemaphoreType.DMA,
            pltpu.SemaphoreType.DMA,
        ],
    )
    def sc_hist(bat_hbm, packed_hbm, out_hbm,
                bat_v, pck_v, idx_v, ones_v, zeros_v, bna_v, bnb_v, hist_sp,
                sem_s, sem_o, sem_z):
        cid = lax.axis_index("c")
        sid = lax.axis_index("s")
        wid = cid * _NS + sid
        base = wid * _CHUNK
        toff = sid * _TS

        def _zfill(i, _):
            zeros_v[pl.ds(i * 16, 16)] = jnp.zeros((16,), jnp.float32)
            return _
        lax.fori_loop(0, _TSQ // 16, _zfill, None)
        for j in range(8):
            ones_v[pl.ds(j * 16, 16)] = jnp.ones((16,), jnp.float32)
        for j in range(1, 8):
            idx_v[_G - 1, pl.ds(j * 16, 16)] = jnp.full((16,), _VP - 1, jnp.int32)

        pltpu.sync_copy(bat_hbm.at[pl.ds(base, _CHUNK)], bat_v)
        pltpu.sync_copy(packed_hbm.at[pl.ds(base, _CHUNK)], pck_v)
        zd = [pltpu.async_copy(zeros_v, hist_sp.at[pl.ds(toff + q * _TSQ, _TSQ)],
                               sem_z) for q in range(_NQ)]
        for d in zd:
            d.wait()
        plsc.subcore_barrier()

        for k in range(3):
            sh = 8 * k

            def _fill(g, _):
                nb = g * 128
                for j in range(8):
                    b16 = bat_v[pl.ds(nb + j * 16, 16)]
                    p16 = pck_v[pl.ds(nb + j * 16, 16)]
                    a16 = lax.shift_right_logical(p16, sh) & 255
                    idx_v[g, pl.ds(j * 16, 16)] = b16 * _VP + a16
                return _
            lax.fori_loop(0, _G - 1, _fill, None)
            b16 = bat_v[pl.ds((_G - 1) * 128, 16)]
            p16 = pck_v[pl.ds((_G - 1) * 128, 16)]
            a16 = lax.shift_right_logical(p16, sh) & 255
            idx_v[_G - 1, pl.ds(0, 16)] = b16 * _VP + a16

            sd = [pltpu.async_copy(ones_v, hist_sp.at[idx_v.at[g]],
                                   sem_s, add=True) for g in range(_G)]
            for d in sd:
                d.wait()

            plsc.subcore_barrier()
            row = cid * 3 + k
            bn = (bna_v, bnb_v)
            wd = [None, None]
            zd = []
            for q in range(_NQ):
                off = toff + q * _TSQ
                b = bn[q & 1]
                if wd[q & 1] is not None:
                    wd[q & 1].wait()
                pltpu.sync_copy(hist_sp.at[pl.ds(off, _TSQ)], b)
                wd[q & 1] = pltpu.async_copy(
                    b, out_hbm.at[pl.ds(row * _SEG_VP + off, _TSQ)], sem_o)
                if k < 2:
                    zd.append(pltpu.async_copy(
                        zeros_v, hist_sp.at[pl.ds(off, _TSQ)], sem_z))
            for d in wd:
                if d is not None:
                    d.wait()
            for d in zd:
                d.wait()
            plsc.subcore_barrier()

    return sc_hist


_BLK = 1024


def _tc_mlp_body(c_ref, e_ref, w1_ref, b1_ref, w2_ref, b2_ref, out_ref):
    C = c_ref[...]
    E = e_ref[...]
    acc = lax.dot(C[0], E[0], preferred_element_type=jnp.float32)
    for i in range(1, 6):
        acc += lax.dot(C[i], E[i], preferred_element_type=jnp.float32)
    vmask = lax.broadcasted_iota(jnp.int32, (_BLK, _VP), 1) < _V
    cnt = jnp.where(vmask, C[0] + C[3], 0.0)
    n = jnp.sum(cnt, axis=1)
    mean = acc / jnp.maximum(n, 1.0)[:, None]
    h = lax.dot_general(mean, w1_ref[...],
                        (((1,), (1,)), ((), ())),
                        preferred_element_type=jnp.float32)
    h = jnp.maximum(h + b1_ref[...][None, :], 0.0)
    o = lax.dot_general(h, w2_ref[...],
                        (((1,), (1,)), ((), ())),
                        preferred_element_type=jnp.float32)
    out_ref[...] = o[:, 0] + b2_ref[0]


def _tc_mlp(C6, E6, W1, b1, W2, b2):
    grid = (_NUM_SEG + _BLK - 1) // _BLK
    return pl.pallas_call(
        _tc_mlp_body,
        grid=(grid,),
        in_specs=[
            pl.BlockSpec((6, _BLK, _VP), lambda i: (0, i, 0)),
            pl.BlockSpec((6, _VP, _H), lambda i: (0, 0, 0)),
            pl.BlockSpec((_H, _H), lambda i: (0, 0)),
            pl.BlockSpec((_H,), lambda i: (0,)),
            pl.BlockSpec((1, _H), lambda i: (0, 0)),
            pl.BlockSpec(memory_space=pltpu.SMEM),
        ],
        out_specs=pl.BlockSpec((_BLK,), lambda i: (i,)),
        out_shape=jax.ShapeDtypeStruct((_NUM_SEG,), jnp.float32),
    )(C6, E6, W1, b1, W2, b2)


def kernel(node, batch_index, emb0, emb1, emb2, W1, b1, W2, b2):
    packed = node[:, 0] + node[:, 1] * 256 + node[:, 2] * 65536
    outC = _sc_histogram_build()(batch_index, packed)
    C6 = outC.reshape(6, _NUM_SEG, _VP)
    E3 = jnp.concatenate(
        [jnp.stack([emb0, emb1, emb2]) * (1.0 / 3.0),
         jnp.zeros((3, _VP - _V, _H), jnp.float32)], axis=1)
    E6 = jnp.concatenate([E3, E3], axis=0)
    return _tc_mlp(C6, E6, W1, b1, W2, b2)


# speedup vs baseline: 33.3149x; 1.0090x over previous
import functools

import jax
import jax.numpy as jnp
from jax import lax
from jax.experimental import pallas as pl
from jax.experimental.pallas import tpu as pltpu
from jax.experimental.pallas import tpu_sc as plsc

_N = 320000
_H = 128
_V = 100
_VP = 128
_NUM_SEG = 10000
_SEG_VP = _NUM_SEG * _VP
_NC = 2
_NS = 16
_NW = _NC * _NS
_CHUNK = _N // _NW
_TS = _SEG_VP // _NS
_TSQ = 4000
_NQ = _TS // _TSQ
_G = 79


@functools.lru_cache(maxsize=1)
def _sc_histogram_build():
    mesh = plsc.VectorSubcoreMesh(core_axis_name="c", subcore_axis_name="s")

    @functools.partial(
        pl.kernel,
        out_type=jax.ShapeDtypeStruct((6 * _SEG_VP,), jnp.float32),
        mesh=mesh,
        scratch_types=[
            pltpu.VMEM((_CHUNK,), jnp.int32),
            pltpu.VMEM((_CHUNK,), jnp.int32),
            pltpu.VMEM((_G, 128), jnp.int32),
            pltpu.VMEM((128,), jnp.float32),
            pltpu.VMEM((_TSQ,), jnp.float32),
            pltpu.VMEM((_TSQ,), jnp.float32),
            pltpu.VMEM((_TSQ,), jnp.float32),
            pltpu.VMEM((_TSQ,), jnp.float32),
            pltpu.VMEM((_TSQ,), jnp.float32),
            pltpu.VMEM_SHARED((_SEG_VP,), jnp.float32),
            pltpu.SemaphoreType.DMA,
            pltpu.SemaphoreType.DMA,
            pltpu.SemaphoreType.DMA,
            pltpu.SemaphoreType.DMA,
        ],
    )
    def sc_hist(bat_hbm, packed_hbm, out_hbm,
                bat_v, pck_v, idx_v, ones_v, zeros_v,
                bn0_v, bn1_v, bn2_v, bn3_v, hist_sp,
                sem_s, sem_o, sem_r, sem_z):
        cid = lax.axis_index("c")
        sid = lax.axis_index("s")
        wid = cid * _NS + sid
        base = wid * _CHUNK
        toff = sid * _TS

        def _zfill(i, _):
            zeros_v[pl.ds(i * 16, 16)] = jnp.zeros((16,), jnp.float32)
            return _
        lax.fori_loop(0, _TSQ // 16, _zfill, None)
        for j in range(8):
            ones_v[pl.ds(j * 16, 16)] = jnp.ones((16,), jnp.float32)
        for j in range(1, 8):
            idx_v[_G - 1, pl.ds(j * 16, 16)] = jnp.full((16,), _VP - 1, jnp.int32)

        pltpu.sync_copy(bat_hbm.at[pl.ds(base, _CHUNK)], bat_v)
        pltpu.sync_copy(packed_hbm.at[pl.ds(base, _CHUNK)], pck_v)
        zd = [pltpu.async_copy(zeros_v, hist_sp.at[pl.ds(toff + q * _TSQ, _TSQ)],
                               sem_z) for q in range(_NQ)]
        for d in zd:
            d.wait()
        plsc.subcore_barrier()

        for k in range(3):
            sh = 8 * k

            def _fill(g, _):
                nb = g * 128
                for j in range(8):
                    b16 = bat_v[pl.ds(nb + j * 16, 16)]
                    p16 = pck_v[pl.ds(nb + j * 16, 16)]
                    a16 = lax.shift_right_logical(p16, sh) & 255
                    idx_v[g, pl.ds(j * 16, 16)] = b16 * _VP + a16
                return _
            lax.fori_loop(0, _G - 1, _fill, None)
            b16 = bat_v[pl.ds((_G - 1) * 128, 16)]
            p16 = pck_v[pl.ds((_G - 1) * 128, 16)]
            a16 = lax.shift_right_logical(p16, sh) & 255
            idx_v[_G - 1, pl.ds(0, 16)] = b16 * _VP + a16

            sd = [pltpu.async_copy(ones_v, hist_sp.at[idx_v.at[g]],
                                   sem_s, add=True) for g in range(_G)]
            for d in sd:
                d.wait()

            plsc.subcore_barrier()
            row = cid * 3 + k
            bn = (bn0_v, bn1_v, bn2_v, bn3_v)
            rd = [None] * 4
            wd = [None] * 4
            zd = []
            for s in range(4):
                rd[s] = pltpu.async_copy(
                    hist_sp.at[pl.ds(toff + s * _TSQ, _TSQ)], bn[s], sem_r)
            for q in range(_NQ):
                s = q & 3
                off = toff + q * _TSQ
                rd[s].wait()
                wd[s] = pltpu.async_copy(
                    bn[s], out_hbm.at[pl.ds(row * _SEG_VP + off, _TSQ)],
                    sem_o)
                if k < 2:
                    zd.append(pltpu.async_copy(
                        zeros_v, hist_sp.at[pl.ds(off, _TSQ)], sem_z))
                if q + 4 < _NQ:
                    wd[s].wait()
                    rd[s] = pltpu.async_copy(
                        hist_sp.at[pl.ds(toff + (q + 4) * _TSQ, _TSQ)],
                        bn[s], sem_r)
            for s in range(4):
                if _NQ - 4 + s >= 0:
                    wd[(_NQ - 4 + s) & 3].wait()
            for d in zd:
                d.wait()
            plsc.subcore_barrier()

    return sc_hist


_BLK = 2048


def _tc_mlp_body(c_ref, e_ref, w1_ref, b1_ref, w2_ref, b2_ref, out_ref):
    C = c_ref[...]
    E = e_ref[...]
    acc = lax.dot(C[0], E[0], preferred_element_type=jnp.float32)
    for i in range(1, 6):
        acc += lax.dot(C[i], E[i], preferred_element_type=jnp.float32)
    vmask = lax.broadcasted_iota(jnp.int32, (_BLK, _VP), 1) < _V
    cnt = jnp.where(vmask, C[0] + C[3], 0.0)
    n = jnp.sum(cnt, axis=1)
    mean = acc / jnp.maximum(n, 1.0)[:, None]
    h = lax.dot_general(mean, w1_ref[...],
                        (((1,), (1,)), ((), ())),
                        preferred_element_type=jnp.float32)
    h = jnp.maximum(h + b1_ref[...][None, :], 0.0)
    o = lax.dot_general(h, w2_ref[...],
                        (((1,), (1,)), ((), ())),
                        preferred_element_type=jnp.float32)
    out_ref[...] = o[:, 0] + b2_ref[0]


def _tc_mlp(C6, E6, W1, b1, W2, b2):
    grid = (_NUM_SEG + _BLK - 1) // _BLK
    return pl.pallas_call(
        _tc_mlp_body,
        grid=(grid,),
        in_specs=[
            pl.BlockSpec((6, _BLK, _VP), lambda i: (0, i, 0)),
            pl.BlockSpec((6, _VP, _H), lambda i: (0, 0, 0)),
            pl.BlockSpec((_H, _H), lambda i: (0, 0)),
            pl.BlockSpec((_H,), lambda i: (0,)),
            pl.BlockSpec((1, _H), lambda i: (0, 0)),
            pl.BlockSpec(memory_space=pltpu.SMEM),
        ],
        out_specs=pl.BlockSpec((_BLK,), lambda i: (i,)),
        out_shape=jax.ShapeDtypeStruct((_NUM_SEG,), jnp.float32),
    )(C6, E6, W1, b1, W2, b2)


def kernel(node, batch_index, emb0, emb1, emb2, W1, b1, W2, b2):
    packed = node[:, 0] + node[:, 1] * 256 + node[:, 2] * 65536
    outC = _sc_histogram_build()(batch_index, packed)
    C6 = outC.reshape(6, _NUM_SEG, _VP)
    E3 = jnp.concatenate(
        [jnp.stack([emb0, emb1, emb2]) * (1.0 / 3.0),
         jnp.zeros((3, _VP - _V, _H), jnp.float32)], axis=1)
    E6 = jnp.concatenate([E3, E3], axis=0)
    return _tc_mlp(C6, E6, W1, b1, W2, b2)
